# baseline (speedup 1.0000x reference)
import jax
import jax.numpy as jnp
from jax.experimental import pallas as pl


def _final_linear_body(p_ref, w_ref, b_ref, o_ref):
    o_ref[...] = jnp.dot(p_ref[...], w_ref[...],
                         preferred_element_type=jnp.float32) + b_ref[...]


def _gcn_conv(x, src, dst, W, b):
    n = x.shape[0]
    loop = jnp.arange(n, dtype=src.dtype)
    src2 = jnp.concatenate([src, loop])
    dst2 = jnp.concatenate([dst, loop])
    h = x @ W
    deg = jax.ops.segment_sum(jnp.ones_like(dst2, dtype=h.dtype), dst2, num_segments=n)
    dinv = jnp.where(deg > 0, jax.lax.rsqrt(deg), 0.0)
    norm = dinv[src2] * dinv[dst2]
    msg = h[src2] * norm[:, None]
    out = jax.ops.segment_sum(msg, dst2, num_segments=n)
    return out + b


def _prelu(x, a):
    return jnp.where(x >= 0, x, a * x)


def kernel(x, edge_index, batch, W1, b1, W2, b2, W3, b3, W4, b4, a1, a2, a3, Wl, bl):
    src, dst = edge_index[0], edge_index[1]
    h = _prelu(_gcn_conv(x, src, dst, W1, b1), a1)
    h = _prelu(_gcn_conv(h, src, dst, W2, b2), a2)
    h = _prelu(_gcn_conv(h, src, dst, W3, b3), a3)
    h = _gcn_conv(h, src, dst, W4, b4)
    G = 64
    sums = jax.ops.segment_sum(h, batch, num_segments=G)
    cnt = jax.ops.segment_sum(jnp.ones((h.shape[0],), h.dtype), batch, num_segments=G)
    pooled = sums / jnp.clip(cnt, 1.0, None)[:, None]
    out = pl.pallas_call(
        _final_linear_body,
        out_shape=jax.ShapeDtypeStruct((G, Wl.shape[1]), jnp.float32),
    )(pooled, Wl, bl[None, :])
    return out


# speedup vs baseline: 14.1150x; 14.1150x over previous
import functools

import jax
import jax.numpy as jnp
from jax import lax
from jax.experimental import pallas as pl
from jax.experimental.pallas import tpu as pltpu
from jax.experimental.pallas import tpu_sc as plsc

N = 10000
E = 320000
G = 64
FW = 128

NW = 32
EW = E // NW
N_PAD = 10240
ROWS_PT = N_PAD // 16
R = 512
NBLK = N_PAD // R
B = 200


def _mesh():
    return plsc.VectorSubcoreMesh(core_axis_name="c", subcore_axis_name="s")


def _agg_partials(ht, src, dst):
    nblk = EW // B

    @functools.partial(
        pl.kernel,
        mesh=_mesh(),
        out_type=(jax.ShapeDtypeStruct((N_PAD, FW), jnp.float32),
                  jax.ShapeDtypeStruct((N_PAD, FW), jnp.float32)),
        scratch_types=[
            pltpu.VMEM((B,), jnp.int32),
            pltpu.VMEM((B,), jnp.int32),
            pltpu.VMEM((B, FW), jnp.float32),
            pltpu.VMEM_SHARED((N_PAD, FW), jnp.float32),
            pltpu.SemaphoreType.DMA,
        ],
    )
    def k(h_hbm, src_hbm, dst_hbm, p0_hbm, p1_hbm,
          sidx, didx, rows, acc_sh, sem):
        c = lax.axis_index("c")
        s = lax.axis_index("s")
        w = s * 2 + c
        r0 = s * ROWS_PT

        @pl.when(c == 0)
        def _():
            pltpu.sync_copy(h_hbm.at[pl.ds(r0, ROWS_PT)],
                            acc_sh.at[pl.ds(r0, ROWS_PT)])

        @pl.when(c == 1)
        def _():
            def zrow(i, _):
                for j in range(FW // 16):
                    rows[i, pl.ds(j * 16, 16)] = jnp.zeros((16,), jnp.float32)
                return 0

            lax.fori_loop(0, B, zrow, 0)
            off = 0
            while off < ROWS_PT:
                sz = min(B, ROWS_PT - off)
                pltpu.sync_copy(rows.at[pl.ds(0, sz)],
                                acc_sh.at[pl.ds(r0 + off, sz)])
                off += sz

        plsc.subcore_barrier()

        def step(i, _):
            base = w * EW + i * B
            pltpu.sync_copy(src_hbm.at[pl.ds(base, B)], sidx)
            pltpu.sync_copy(dst_hbm.at[pl.ds(base, B)], didx)
            pltpu.async_copy(h_hbm.at[sidx], rows, sem).wait()
            pltpu.sync_copy(rows, acc_sh.at[didx], add=True)
            return 0

        lax.fori_loop(0, nblk, step, 0)
        plsc.subcore_barrier()

        @pl.when(c == 0)
        def _():
            pltpu.sync_copy(acc_sh.at[pl.ds(r0, ROWS_PT)],
                            p0_hbm.at[pl.ds(r0, ROWS_PT)])

        @pl.when(c == 1)
        def _():
            pltpu.sync_copy(acc_sh.at[pl.ds(r0, ROWS_PT)],
                            p1_hbm.at[pl.ds(r0, ROWS_PT)])

    return k(ht, src, dst)


def _deg_partials(dst):
    Bd = 200
    nblk = EW // Bd

    @functools.partial(
        pl.kernel,
        mesh=_mesh(),
        out_type=(jax.ShapeDtypeStruct((N_PAD, FW), jnp.float32),
                  jax.ShapeDtypeStruct((N_PAD, FW), jnp.float32)),
        scratch_types=[
            pltpu.VMEM((Bd,), jnp.int32),
            pltpu.VMEM((Bd, FW), jnp.float32),
            pltpu.VMEM_SHARED((N_PAD, FW), jnp.float32),
        ],
    )
    def k(dst_hbm, p0_hbm, p1_hbm, idx_v, ones_v, acc_sh):
        c = lax.axis_index("c")
        s = lax.axis_index("s")
        w = s * 2 + c
        r0 = s * ROWS_PT

        def fill(i, _):
            for j in range(FW // 16):
                ones_v[i, pl.ds(j * 16, 16)] = jnp.zeros((16,), jnp.float32)
            return 0

        lax.fori_loop(0, Bd, fill, 0)
        off = 0
        while off < ROWS_PT:
            sz = min(Bd, ROWS_PT - off)
            pltpu.sync_copy(ones_v.at[pl.ds(0, sz)],
                            acc_sh.at[pl.ds(r0 + off, sz)])
            off += sz

        def fill1(i, _):
            for j in range(FW // 16):
                ones_v[i, pl.ds(j * 16, 16)] = jnp.full((16,), 1.0,
                                                        jnp.float32)
            return 0

        lax.fori_loop(0, Bd, fill1, 0)
        plsc.subcore_barrier()

        def step(i, _):
            pltpu.sync_copy(dst_hbm.at[pl.ds(w * EW + i * Bd, Bd)], idx_v)
            pltpu.sync_copy(ones_v, acc_sh.at[idx_v], add=True)
            return 0

        lax.fori_loop(0, nblk, step, 0)
        plsc.subcore_barrier()

        @pl.when(c == 0)
        def _():
            pltpu.sync_copy(acc_sh.at[pl.ds(r0, ROWS_PT)],
                            p0_hbm.at[pl.ds(r0, ROWS_PT)])

        @pl.when(c == 1)
        def _():
            pltpu.sync_copy(acc_sh.at[pl.ds(r0, ROWS_PT)],
                            p1_hbm.at[pl.ds(r0, ROWS_PT)])

    return k(dst)


def _pad_cols(v, width):
    if v.shape[1] == width:
        return v
    return jnp.concatenate(
        [v, jnp.zeros((v.shape[0], width - v.shape[1]), v.dtype)], axis=1)


def _tc_first(x_pad, d0, d1, W1):
    F = W1.shape[1]

    def body(x_ref, d0_ref, d1_ref, w_ref, dv_ref, ht_ref):
        deg = d0_ref[...] + d1_ref[...] + 1.0
        dv = lax.rsqrt(deg)
        dv_ref[...] = dv
        ht = jnp.dot(x_ref[...], w_ref[...],
                     preferred_element_type=jnp.float32)
        ht_ref[...] = _pad_cols(ht, FW) * dv

    return pl.pallas_call(
        body,
        grid=(NBLK,),
        in_specs=[
            pl.BlockSpec((R, 128), lambda i: (i, 0)),
            pl.BlockSpec((R, FW), lambda i: (i, 0)),
            pl.BlockSpec((R, FW), lambda i: (i, 0)),
            pl.BlockSpec(W1.shape, lambda i: (0, 0)),
        ],
        out_specs=[
            pl.BlockSpec((R, FW), lambda i: (i, 0)),
            pl.BlockSpec((R, FW), lambda i: (i, 0)),
        ],
        out_shape=[
            jax.ShapeDtypeStruct((N_PAD, FW), jnp.float32),
            jax.ShapeDtypeStruct((N_PAD, FW), jnp.float32),
        ],
    )(x_pad, d0, d1, W1)


def _tc_mid(p0, p1, dvf, b_prev, a_prev, W_next):
    Fp = W_next.shape[0]
    Fn = W_next.shape[1]
    b_pad = jnp.zeros((1, FW), jnp.float32).at[0, :Fp].set(b_prev)
    W_pad = jnp.zeros((FW, Fn), jnp.float32).at[:Fp].set(W_next)

    def body(p0_ref, p1_ref, dv_ref, b_ref, a_ref, w_ref, ht_ref):
        dv = dv_ref[...]
        t = dv * (p0_ref[...] + p1_ref[...]) + b_ref[...]
        a = a_ref[0, 0]
        h = jnp.where(t >= 0, t, a * t)
        ht = jnp.dot(h, w_ref[...], preferred_element_type=jnp.float32)
        ht_ref[...] = _pad_cols(ht, FW) * dv

    return pl.pallas_call(
        body,
        grid=(NBLK,),
        in_specs=[
            pl.BlockSpec((R, FW), lambda i: (i, 0)),
            pl.BlockSpec((R, FW), lambda i: (i, 0)),
            pl.BlockSpec((R, FW), lambda i: (i, 0)),
            pl.BlockSpec((1, FW), lambda i: (0, 0)),
            pl.BlockSpec(memory_space=pltpu.SMEM),
            pl.BlockSpec((FW, Fn), lambda i: (0, 0)),
        ],
        out_specs=pl.BlockSpec((R, FW), lambda i: (i, 0)),
        out_shape=jax.ShapeDtypeStruct((N_PAD, FW), jnp.float32),
    )(p0, p1, dvf, b_pad, a_prev.reshape(1, 1), W_pad)


def _tc_last(p0, p1, dvf, b4, batch_col, Wl, bl):
    C = Wl.shape[1]

    def body(p0_ref, p1_ref, dv_ref, b_ref, bat_ref, wl_ref, bl_ref,
             out_ref, sums, cnt):
        i = pl.program_id(0)

        @pl.when(i == 0)
        def _():
            sums[...] = jnp.zeros_like(sums)
            cnt[...] = jnp.zeros_like(cnt)

        h4 = dv_ref[...] * (p0_ref[...] + p1_ref[...]) + b_ref[...]
        seg = lax.broadcasted_iota(jnp.int32, (R, G), 1)
        onehot = (seg == bat_ref[...]).astype(jnp.float32)
        dn = (((0,), (0,)), ((), ()))
        sums[...] += lax.dot_general(onehot, h4, dn,
                                     preferred_element_type=jnp.float32)
        cnt[...] += lax.dot_general(onehot, jnp.full((R, 1), 1.0,
                                                     jnp.float32), dn,
                                    preferred_element_type=jnp.float32)

        @pl.when(i == NBLK - 1)
        def _():
            pooled = sums[...] / jnp.clip(cnt[...], 1.0, None)
            out_ref[...] = jnp.dot(pooled, wl_ref[...],
                                   preferred_element_type=jnp.float32) + bl_ref[...]

    return pl.pallas_call(
        body,
        grid=(NBLK,),
        in_specs=[
            pl.BlockSpec((R, FW), lambda i: (i, 0)),
            pl.BlockSpec((R, FW), lambda i: (i, 0)),
            pl.BlockSpec((R, FW), lambda i: (i, 0)),
            pl.BlockSpec((1, FW), lambda i: (0, 0)),
            pl.BlockSpec((R, 1), lambda i: (i, 0)),
            pl.BlockSpec(Wl.shape, lambda i: (0, 0)),
            pl.BlockSpec((1, C), lambda i: (0, 0)),
        ],
        out_specs=pl.BlockSpec((G, C), lambda i: (0, 0)),
        out_shape=jax.ShapeDtypeStruct((G, C), jnp.float32),
        scratch_shapes=[
            pltpu.VMEM((G, FW), jnp.float32),
            pltpu.VMEM((G, 1), jnp.float32),
        ],
    )(p0, p1, dvf, b4.reshape(1, FW), batch_col, Wl, bl.reshape(1, C))


def kernel(x, edge_index, batch, W1, b1, W2, b2, W3, b3, W4, b4,
           a1, a2, a3, Wl, bl):
    src = edge_index[0]
    dst = edge_index[1]
    x_pad = jnp.zeros((N_PAD, 128), jnp.float32).at[:N].set(x)
    batch_pad = jnp.full((N_PAD,), G, jnp.int32).at[:N].set(batch)
    batch_col = batch_pad.reshape(N_PAD, 1)

    d0, d1 = _deg_partials(dst)
    dvf, ht1 = _tc_first(x_pad, d0, d1, W1)

    p0, p1 = _agg_partials(ht1, src, dst)
    ht2 = _tc_mid(p0, p1, dvf, b1, a1, W2)

    p0, p1 = _agg_partials(ht2, src, dst)
    ht3 = _tc_mid(p0, p1, dvf, b2, a2, W3)

    p0, p1 = _agg_partials(ht3, src, dst)
    ht4 = _tc_mid(p0, p1, dvf, b3, a3, W4)

    p0, p1 = _agg_partials(ht4, src, dst)
    return _tc_last(p0, p1, dvf, b4, batch_col, Wl, bl)


# speedup vs baseline: 16.9862x; 1.2034x over previous
import functools

import jax
import jax.numpy as jnp
from jax import lax
from jax.experimental import pallas as pl
from jax.experimental.pallas import tpu as pltpu
from jax.experimental.pallas import tpu_sc as plsc

N = 10000
E = 320000
G = 64
FW = 128

NW = 32
EW = E // NW
N_PAD = 10240
ROWS_PT = N_PAD // 16
R = 512
NBLK = N_PAD // R
B = 184
NFULL = 54
TAIL = EW - NFULL * B


def _mesh():
    return plsc.VectorSubcoreMesh(core_axis_name="c", subcore_axis_name="s")


def _agg_partials(ht, src, dst):
    @functools.partial(
        pl.kernel,
        mesh=_mesh(),
        out_type=(jax.ShapeDtypeStruct((N_PAD, FW), jnp.float32),
                  jax.ShapeDtypeStruct((N_PAD, FW), jnp.float32)),
        scratch_types=[
            pltpu.VMEM((B,), jnp.int32),
            pltpu.VMEM((B,), jnp.int32),
            pltpu.VMEM((B, FW), jnp.float32),
            pltpu.VMEM((B,), jnp.int32),
            pltpu.VMEM((B,), jnp.int32),
            pltpu.VMEM((B, FW), jnp.float32),
            pltpu.VMEM((TAIL,), jnp.int32),
            pltpu.VMEM((TAIL,), jnp.int32),
            pltpu.VMEM_SHARED((N_PAD, FW), jnp.float32),
            pltpu.SemaphoreType.DMA,
            pltpu.SemaphoreType.DMA,
            pltpu.SemaphoreType.DMA,
        ],
    )
    def k(h_hbm, src_hbm, dst_hbm, p0_hbm, p1_hbm,
          sidx0, didx0, rows0, sidx1, didx1, rows1, sidx_t, didx_t,
          acc_sh, gsem, ssem0, ssem1):
        c = lax.axis_index("c")
        s = lax.axis_index("s")
        w = s * 2 + c
        r0 = s * ROWS_PT
        sidx = [sidx0, sidx1]
        didx = [didx0, didx1]
        rows = [rows0, rows1]
        ssem = [ssem0, ssem1]

        @pl.when(c == 0)
        def _():
            pltpu.sync_copy(h_hbm.at[pl.ds(r0, ROWS_PT)],
                            acc_sh.at[pl.ds(r0, ROWS_PT)])

        @pl.when(c == 1)
        def _():
            def zrow(i, _):
                for j in range(FW // 16):
                    rows0[i, pl.ds(j * 16, 16)] = jnp.zeros((16,),
                                                            jnp.float32)
                return 0

            lax.fori_loop(0, B, zrow, 0)
            off = 0
            while off < ROWS_PT:
                sz = min(B, ROWS_PT - off)
                pltpu.sync_copy(rows0.at[pl.ds(0, sz)],
                                acc_sh.at[pl.ds(r0 + off, sz)])
                off += sz

        plsc.subcore_barrier()

        def do_block(i, b, first):
            if not first:
                pltpu.make_async_copy(rows[b], acc_sh.at[didx[b]],
                                      ssem[b]).wait()
            base = w * EW + i * B
            pltpu.sync_copy(src_hbm.at[pl.ds(base, B)], sidx[b])
            pltpu.sync_copy(dst_hbm.at[pl.ds(base, B)], didx[b])
            pltpu.async_copy(h_hbm.at[sidx[b]], rows[b], gsem).wait()
            pltpu.async_copy(rows[b], acc_sh.at[didx[b]], ssem[b], add=True)

        do_block(0, 0, True)
        do_block(1, 1, True)

        def pair(j, _):
            do_block(2 * j, 0, False)
            do_block(2 * j + 1, 1, False)
            return 0

        lax.fori_loop(1, NFULL // 2, pair, 0)
        pltpu.make_async_copy(rows0, acc_sh.at[didx0], ssem0).wait()
        pltpu.make_async_copy(rows1, acc_sh.at[didx1], ssem1).wait()

        tbase = w * EW + NFULL * B
        pltpu.sync_copy(src_hbm.at[pl.ds(tbase, TAIL)], sidx_t)
        pltpu.sync_copy(dst_hbm.at[pl.ds(tbase, TAIL)], didx_t)
        pltpu.async_copy(h_hbm.at[sidx_t], rows0.at[pl.ds(0, TAIL)],
                         gsem).wait()
        pltpu.sync_copy(rows0.at[pl.ds(0, TAIL)], acc_sh.at[didx_t],
                        add=True)
        plsc.subcore_barrier()

        @pl.when(c == 0)
        def _():
            pltpu.sync_copy(acc_sh.at[pl.ds(r0, ROWS_PT)],
                            p0_hbm.at[pl.ds(r0, ROWS_PT)])

        @pl.when(c == 1)
        def _():
            pltpu.sync_copy(acc_sh.at[pl.ds(r0, ROWS_PT)],
                            p1_hbm.at[pl.ds(r0, ROWS_PT)])

    return k(ht, src, dst)


def _deg_partials(dst):
    Bd = 200
    nblk = EW // Bd

    @functools.partial(
        pl.kernel,
        mesh=_mesh(),
        out_type=(jax.ShapeDtypeStruct((N_PAD, FW), jnp.float32),
                  jax.ShapeDtypeStruct((N_PAD, FW), jnp.float32)),
        scratch_types=[
            pltpu.VMEM((Bd,), jnp.int32),
            pltpu.VMEM((Bd,), jnp.int32),
            pltpu.VMEM((Bd, FW), jnp.float32),
            pltpu.VMEM_SHARED((N_PAD, FW), jnp.float32),
            pltpu.SemaphoreType.DMA,
            pltpu.SemaphoreType.DMA,
        ],
    )
    def k(dst_hbm, p0_hbm, p1_hbm, idx0, idx1, ones_v, acc_sh,
          ssem0, ssem1):
        c = lax.axis_index("c")
        s = lax.axis_index("s")
        w = s * 2 + c
        r0 = s * ROWS_PT
        idx = [idx0, idx1]
        ssem = [ssem0, ssem1]

        def fill(i, _):
            for j in range(FW // 16):
                ones_v[i, pl.ds(j * 16, 16)] = jnp.zeros((16,), jnp.float32)
            return 0

        lax.fori_loop(0, Bd, fill, 0)
        off = 0
        while off < ROWS_PT:
            sz = min(Bd, ROWS_PT - off)
            pltpu.sync_copy(ones_v.at[pl.ds(0, sz)],
                            acc_sh.at[pl.ds(r0 + off, sz)])
            off += sz

        def fill1(i, _):
            for j in range(FW // 16):
                ones_v[i, pl.ds(j * 16, 16)] = jnp.full((16,), 1.0,
                                                        jnp.float32)
            return 0

        lax.fori_loop(0, Bd, fill1, 0)
        plsc.subcore_barrier()

        def do_block(i, b, first):
            if not first:
                pltpu.make_async_copy(ones_v, acc_sh.at[idx[b]],
                                      ssem[b]).wait()
            pltpu.sync_copy(dst_hbm.at[pl.ds(w * EW + i * Bd, Bd)], idx[b])
            pltpu.async_copy(ones_v, acc_sh.at[idx[b]], ssem[b], add=True)

        do_block(0, 0, True)
        do_block(1, 1, True)

        def pair(j, _):
            do_block(2 * j, 0, False)
            do_block(2 * j + 1, 1, False)
            return 0

        lax.fori_loop(1, nblk // 2, pair, 0)
        pltpu.make_async_copy(ones_v, acc_sh.at[idx0], ssem0).wait()
        pltpu.make_async_copy(ones_v, acc_sh.at[idx1], ssem1).wait()
        plsc.subcore_barrier()

        @pl.when(c == 0)
        def _():
            pltpu.sync_copy(acc_sh.at[pl.ds(r0, ROWS_PT)],
                            p0_hbm.at[pl.ds(r0, ROWS_PT)])

        @pl.when(c == 1)
        def _():
            pltpu.sync_copy(acc_sh.at[pl.ds(r0, ROWS_PT)],
                            p1_hbm.at[pl.ds(r0, ROWS_PT)])

    return k(dst)


def _pad_cols(v, width):
    if v.shape[1] == width:
        return v
    return jnp.concatenate(
        [v, jnp.zeros((v.shape[0], width - v.shape[1]), v.dtype)], axis=1)


def _tc_first(x_pad, d0, d1, W1):
    F = W1.shape[1]

    def body(x_ref, d0_ref, d1_ref, w_ref, dv_ref, ht_ref):
        deg = d0_ref[...] + d1_ref[...] + 1.0
        dv = lax.rsqrt(deg)
        dv_ref[...] = dv
        ht = jnp.dot(x_ref[...], w_ref[...],
                     preferred_element_type=jnp.float32)
        ht_ref[...] = _pad_cols(ht, FW) * dv

    return pl.pallas_call(
        body,
        grid=(NBLK,),
        in_specs=[
            pl.BlockSpec((R, 128), lambda i: (i, 0)),
            pl.BlockSpec((R, FW), lambda i: (i, 0)),
            pl.BlockSpec((R, FW), lambda i: (i, 0)),
            pl.BlockSpec(W1.shape, lambda i: (0, 0)),
        ],
        out_specs=[
            pl.BlockSpec((R, FW), lambda i: (i, 0)),
            pl.BlockSpec((R, FW), lambda i: (i, 0)),
        ],
        out_shape=[
            jax.ShapeDtypeStruct((N_PAD, FW), jnp.float32),
            jax.ShapeDtypeStruct((N_PAD, FW), jnp.float32),
        ],
    )(x_pad, d0, d1, W1)


def _tc_mid(p0, p1, dvf, b_prev, a_prev, W_next):
    Fp = W_next.shape[0]
    Fn = W_next.shape[1]
    b_pad = jnp.zeros((1, FW), jnp.float32).at[0, :Fp].set(b_prev)
    W_pad = jnp.zeros((FW, Fn), jnp.float32).at[:Fp].set(W_next)

    def body(p0_ref, p1_ref, dv_ref, b_ref, a_ref, w_ref, ht_ref):
        dv = dv_ref[...]
        t = dv * (p0_ref[...] + p1_ref[...]) + b_ref[...]
        a = a_ref[0, 0]
        h = jnp.where(t >= 0, t, a * t)
        ht = jnp.dot(h, w_ref[...], preferred_element_type=jnp.float32)
        ht_ref[...] = _pad_cols(ht, FW) * dv

    return pl.pallas_call(
        body,
        grid=(NBLK,),
        in_specs=[
            pl.BlockSpec((R, FW), lambda i: (i, 0)),
            pl.BlockSpec((R, FW), lambda i: (i, 0)),
            pl.BlockSpec((R, FW), lambda i: (i, 0)),
            pl.BlockSpec((1, FW), lambda i: (0, 0)),
            pl.BlockSpec(memory_space=pltpu.SMEM),
            pl.BlockSpec((FW, Fn), lambda i: (0, 0)),
        ],
        out_specs=pl.BlockSpec((R, FW), lambda i: (i, 0)),
        out_shape=jax.ShapeDtypeStruct((N_PAD, FW), jnp.float32),
    )(p0, p1, dvf, b_pad, a_prev.reshape(1, 1), W_pad)


def _tc_last(p0, p1, dvf, b4, batch_col, Wl, bl):
    C = Wl.shape[1]

    def body(p0_ref, p1_ref, dv_ref, b_ref, bat_ref, wl_ref, bl_ref,
             out_ref, sums, cnt):
        i = pl.program_id(0)

        @pl.when(i == 0)
        def _():
            sums[...] = jnp.zeros_like(sums)
            cnt[...] = jnp.zeros_like(cnt)

        h4 = dv_ref[...] * (p0_ref[...] + p1_ref[...]) + b_ref[...]
        seg = lax.broadcasted_iota(jnp.int32, (R, G), 1)
        onehot = (seg == bat_ref[...]).astype(jnp.float32)
        dn = (((0,), (0,)), ((), ()))
        sums[...] += lax.dot_general(onehot, h4, dn,
                                     preferred_element_type=jnp.float32)
        cnt[...] += lax.dot_general(onehot, jnp.full((R, 1), 1.0,
                                                     jnp.float32), dn,
                                    preferred_element_type=jnp.float32)

        @pl.when(i == NBLK - 1)
        def _():
            pooled = sums[...] / jnp.clip(cnt[...], 1.0, None)
            out_ref[...] = jnp.dot(pooled, wl_ref[...],
                                   preferred_element_type=jnp.float32) + bl_ref[...]

    return pl.pallas_call(
        body,
        grid=(NBLK,),
        in_specs=[
            pl.BlockSpec((R, FW), lambda i: (i, 0)),
            pl.BlockSpec((R, FW), lambda i: (i, 0)),
            pl.BlockSpec((R, FW), lambda i: (i, 0)),
            pl.BlockSpec((1, FW), lambda i: (0, 0)),
            pl.BlockSpec((R, 1), lambda i: (i, 0)),
            pl.BlockSpec(Wl.shape, lambda i: (0, 0)),
            pl.BlockSpec((1, C), lambda i: (0, 0)),
        ],
        out_specs=pl.BlockSpec((G, C), lambda i: (0, 0)),
        out_shape=jax.ShapeDtypeStruct((G, C), jnp.float32),
        scratch_shapes=[
            pltpu.VMEM((G, FW), jnp.float32),
            pltpu.VMEM((G, 1), jnp.float32),
        ],
    )(p0, p1, dvf, b4.reshape(1, FW), batch_col, Wl, bl.reshape(1, C))


def kernel(x, edge_index, batch, W1, b1, W2, b2, W3, b3, W4, b4,
           a1, a2, a3, Wl, bl):
    src = edge_index[0]
    dst = edge_index[1]
    x_pad = jnp.zeros((N_PAD, 128), jnp.float32).at[:N].set(x)
    batch_pad = jnp.full((N_PAD,), G, jnp.int32).at[:N].set(batch)
    batch_col = batch_pad.reshape(N_PAD, 1)

    d0, d1 = _deg_partials(dst)
    dvf, ht1 = _tc_first(x_pad, d0, d1, W1)

    p0, p1 = _agg_partials(ht1, src, dst)
    ht2 = _tc_mid(p0, p1, dvf, b1, a1, W2)

    p0, p1 = _agg_partials(ht2, src, dst)
    ht3 = _tc_mid(p0, p1, dvf, b2, a2, W3)

    p0, p1 = _agg_partials(ht3, src, dst)
    ht4 = _tc_mid(p0, p1, dvf, b3, a3, W4)

    p0, p1 = _agg_partials(ht4, src, dst)
    return _tc_last(p0, p1, dvf, b4, batch_col, Wl, bl)


# speedup vs baseline: 20.4806x; 1.2057x over previous
import functools

import jax
import jax.numpy as jnp
from jax import lax
from jax.experimental import pallas as pl
from jax.experimental.pallas import tpu as pltpu
from jax.experimental.pallas import tpu_sc as plsc

N = 10000
E = 320000
G = 64
FW = 128

NW = 32
EW = E // NW
N_PAD = 10240
ROWS_PT = N_PAD // 16
R = 512
NBLK = N_PAD // R
B = 120
NFULL = 81
TAIL_SZ = (120, 120, 40)


def _mesh():
    return plsc.VectorSubcoreMesh(core_axis_name="c", subcore_axis_name="s")


def _agg_partials(ht, src, dst):
    @functools.partial(
        pl.kernel,
        mesh=_mesh(),
        out_type=(jax.ShapeDtypeStruct((N_PAD, FW), jnp.float32),
                  jax.ShapeDtypeStruct((N_PAD, FW), jnp.float32)),
        scratch_types=[
            pltpu.VMEM((B,), jnp.int32),
            pltpu.VMEM((B,), jnp.int32),
            pltpu.VMEM((B, FW), jnp.float32),
            pltpu.VMEM((B,), jnp.int32),
            pltpu.VMEM((B,), jnp.int32),
            pltpu.VMEM((B, FW), jnp.float32),
            pltpu.VMEM((B,), jnp.int32),
            pltpu.VMEM((B,), jnp.int32),
            pltpu.VMEM((B, FW), jnp.float32),
            pltpu.VMEM((TAIL_SZ[2],), jnp.int32),
            pltpu.VMEM((TAIL_SZ[2],), jnp.int32),
            pltpu.VMEM_SHARED((N_PAD, FW), jnp.float32),
            pltpu.SemaphoreType.DMA,
            pltpu.SemaphoreType.DMA,
            pltpu.SemaphoreType.DMA,
            pltpu.SemaphoreType.DMA,
            pltpu.SemaphoreType.DMA,
            pltpu.SemaphoreType.DMA,
        ],
    )
    def k(h_hbm, src_hbm, dst_hbm, p0_hbm, p1_hbm,
          sidx0, didx0, rows0, sidx1, didx1, rows1, sidx2, didx2, rows2,
          sidx_t, didx_t, acc_sh,
          gsem0, gsem1, gsem2, ssem0, ssem1, ssem2):
        c = lax.axis_index("c")
        s = lax.axis_index("s")
        w = s * 2 + c
        r0 = s * ROWS_PT
        sidx = [sidx0, sidx1, sidx2]
        didx = [didx0, didx1, didx2]
        rows = [rows0, rows1, rows2]
        gsem = [gsem0, gsem1, gsem2]
        ssem = [ssem0, ssem1, ssem2]

        @pl.when(c == 0)
        def _():
            pltpu.sync_copy(h_hbm.at[pl.ds(r0, ROWS_PT)],
                            acc_sh.at[pl.ds(r0, ROWS_PT)])

        @pl.when(c == 1)
        def _():
            def zrow(i, _):
                for j in range(FW // 16):
                    rows0[i, pl.ds(j * 16, 16)] = jnp.zeros((16,),
                                                            jnp.float32)
                return 0

            lax.fori_loop(0, B, zrow, 0)
            off = 0
            while off < ROWS_PT:
                sz = min(B, ROWS_PT - off)
                pltpu.sync_copy(rows0.at[pl.ds(0, sz)],
                                acc_sh.at[pl.ds(r0 + off, sz)])
                off += sz

        plsc.subcore_barrier()

        def phase_a(i, b, drain):
            if drain:
                pltpu.make_async_copy(rows[b], acc_sh.at[didx[b]],
                                      ssem[b]).wait()
            base = w * EW + i * B
            pltpu.sync_copy(src_hbm.at[pl.ds(base, B)], sidx[b])
            pltpu.sync_copy(dst_hbm.at[pl.ds(base, B)], didx[b])
            pltpu.async_copy(h_hbm.at[sidx[b]], rows[b], gsem[b])

        def phase_b(i, b):
            pltpu.make_async_copy(h_hbm.at[sidx[b]], rows[b],
                                  gsem[b]).wait()
            pltpu.async_copy(rows[b], acc_sh.at[didx[b]], ssem[b], add=True)

        phase_a(0, 0, False)
        phase_a(1, 1, False)
        phase_b(0, 0)
        phase_a(2, 2, False)
        phase_b(1, 1)

        def tri(m, _):
            i0 = 3 * m
            phase_a(i0, 0, True)
            phase_b(i0 - 1, 2)
            phase_a(i0 + 1, 1, True)
            phase_b(i0, 0)
            phase_a(i0 + 2, 2, True)
            phase_b(i0 + 1, 1)
            return 0

        lax.fori_loop(1, NFULL // 3, tri, 0)
        phase_b(NFULL - 1, (NFULL - 1) % 3)
        for b in range(3):
            pltpu.make_async_copy(rows[b], acc_sh.at[didx[b]],
                                  ssem[b]).wait()

        toff = NFULL * B
        tidx = [(sidx0, didx0, rows0), (sidx1, didx1, rows1),
                (sidx_t, didx_t, rows2)]
        for (sz, (sb, db, rb)) in zip(TAIL_SZ, tidx):
            base = w * EW + toff
            pltpu.sync_copy(src_hbm.at[pl.ds(base, sz)], sb.at[pl.ds(0, sz)]
                            if sz != sb.shape[0] else sb)
            pltpu.sync_copy(dst_hbm.at[pl.ds(base, sz)], db.at[pl.ds(0, sz)]
                            if sz != db.shape[0] else db)
            pltpu.async_copy(h_hbm.at[sb if sz == sb.shape[0]
                                      else sb.at[pl.ds(0, sz)]],
                             rb.at[pl.ds(0, sz)], gsem0).wait()
            pltpu.sync_copy(rb.at[pl.ds(0, sz)],
                            acc_sh.at[db if sz == db.shape[0]
                                      else db.at[pl.ds(0, sz)]],
                            add=True)
            toff += sz
        plsc.subcore_barrier()

        @pl.when(c == 0)
        def _():
            pltpu.sync_copy(acc_sh.at[pl.ds(r0, ROWS_PT)],
                            p0_hbm.at[pl.ds(r0, ROWS_PT)])

        @pl.when(c == 1)
        def _():
            pltpu.sync_copy(acc_sh.at[pl.ds(r0, ROWS_PT)],
                            p1_hbm.at[pl.ds(r0, ROWS_PT)])

    return k(ht, src, dst)


def _deg_partials(dst):
    Bd = 200
    nblk = EW // Bd

    @functools.partial(
        pl.kernel,
        mesh=_mesh(),
        out_type=(jax.ShapeDtypeStruct((N_PAD, FW), jnp.float32),
                  jax.ShapeDtypeStruct((N_PAD, FW), jnp.float32)),
        scratch_types=[
            pltpu.VMEM((Bd,), jnp.int32),
            pltpu.VMEM((Bd,), jnp.int32),
            pltpu.VMEM((Bd, FW), jnp.float32),
            pltpu.VMEM_SHARED((N_PAD, FW), jnp.float32),
            pltpu.SemaphoreType.DMA,
            pltpu.SemaphoreType.DMA,
        ],
    )
    def k(dst_hbm, p0_hbm, p1_hbm, idx0, idx1, ones_v, acc_sh,
          ssem0, ssem1):
        c = lax.axis_index("c")
        s = lax.axis_index("s")
        w = s * 2 + c
        r0 = s * ROWS_PT
        idx = [idx0, idx1]
        ssem = [ssem0, ssem1]

        def fill(i, _):
            for j in range(FW // 16):
                ones_v[i, pl.ds(j * 16, 16)] = jnp.zeros((16,), jnp.float32)
            return 0

        lax.fori_loop(0, Bd, fill, 0)
        off = 0
        while off < ROWS_PT:
            sz = min(Bd, ROWS_PT - off)
            pltpu.sync_copy(ones_v.at[pl.ds(0, sz)],
                            acc_sh.at[pl.ds(r0 + off, sz)])
            off += sz

        def fill1(i, _):
            for j in range(FW // 16):
                ones_v[i, pl.ds(j * 16, 16)] = jnp.full((16,), 1.0,
                                                        jnp.float32)
            return 0

        lax.fori_loop(0, Bd, fill1, 0)
        plsc.subcore_barrier()

        def do_block(i, b, first):
            if not first:
                pltpu.make_async_copy(ones_v, acc_sh.at[idx[b]],
                                      ssem[b]).wait()
            pltpu.sync_copy(dst_hbm.at[pl.ds(w * EW + i * Bd, Bd)], idx[b])
            pltpu.async_copy(ones_v, acc_sh.at[idx[b]], ssem[b], add=True)

        do_block(0, 0, True)
        do_block(1, 1, True)

        def pair(j, _):
            do_block(2 * j, 0, False)
            do_block(2 * j + 1, 1, False)
            return 0

        lax.fori_loop(1, nblk // 2, pair, 0)
        pltpu.make_async_copy(ones_v, acc_sh.at[idx0], ssem0).wait()
        pltpu.make_async_copy(ones_v, acc_sh.at[idx1], ssem1).wait()
        plsc.subcore_barrier()

        @pl.when(c == 0)
        def _():
            pltpu.sync_copy(acc_sh.at[pl.ds(r0, ROWS_PT)],
                            p0_hbm.at[pl.ds(r0, ROWS_PT)])

        @pl.when(c == 1)
        def _():
            pltpu.sync_copy(acc_sh.at[pl.ds(r0, ROWS_PT)],
                            p1_hbm.at[pl.ds(r0, ROWS_PT)])

    return k(dst)


def _pad_cols(v, width):
    if v.shape[1] == width:
        return v
    return jnp.concatenate(
        [v, jnp.zeros((v.shape[0], width - v.shape[1]), v.dtype)], axis=1)


def _tc_first(x_pad, d0, d1, W1):
    F = W1.shape[1]

    def body(x_ref, d0_ref, d1_ref, w_ref, dv_ref, ht_ref):
        deg = d0_ref[...] + d1_ref[...] + 1.0
        dv = lax.rsqrt(deg)
        dv_ref[...] = dv
        ht = jnp.dot(x_ref[...], w_ref[...],
                     preferred_element_type=jnp.float32)
        ht_ref[...] = _pad_cols(ht, FW) * dv

    return pl.pallas_call(
        body,
        grid=(NBLK,),
        in_specs=[
            pl.BlockSpec((R, 128), lambda i: (i, 0)),
            pl.BlockSpec((R, FW), lambda i: (i, 0)),
            pl.BlockSpec((R, FW), lambda i: (i, 0)),
            pl.BlockSpec(W1.shape, lambda i: (0, 0)),
        ],
        out_specs=[
            pl.BlockSpec((R, FW), lambda i: (i, 0)),
            pl.BlockSpec((R, FW), lambda i: (i, 0)),
        ],
        out_shape=[
            jax.ShapeDtypeStruct((N_PAD, FW), jnp.float32),
            jax.ShapeDtypeStruct((N_PAD, FW), jnp.float32),
        ],
    )(x_pad, d0, d1, W1)


def _tc_mid(p0, p1, dvf, b_prev, a_prev, W_next):
    Fp = W_next.shape[0]
    Fn = W_next.shape[1]
    b_pad = jnp.zeros((1, FW), jnp.float32).at[0, :Fp].set(b_prev)
    W_pad = jnp.zeros((FW, Fn), jnp.float32).at[:Fp].set(W_next)

    def body(p0_ref, p1_ref, dv_ref, b_ref, a_ref, w_ref, ht_ref):
        dv = dv_ref[...]
        t = dv * (p0_ref[...] + p1_ref[...]) + b_ref[...]
        a = a_ref[0, 0]
        h = jnp.where(t >= 0, t, a * t)
        ht = jnp.dot(h, w_ref[...], preferred_element_type=jnp.float32)
        ht_ref[...] = _pad_cols(ht, FW) * dv

    return pl.pallas_call(
        body,
        grid=(NBLK,),
        in_specs=[
            pl.BlockSpec((R, FW), lambda i: (i, 0)),
            pl.BlockSpec((R, FW), lambda i: (i, 0)),
            pl.BlockSpec((R, FW), lambda i: (i, 0)),
            pl.BlockSpec((1, FW), lambda i: (0, 0)),
            pl.BlockSpec(memory_space=pltpu.SMEM),
            pl.BlockSpec((FW, Fn), lambda i: (0, 0)),
        ],
        out_specs=pl.BlockSpec((R, FW), lambda i: (i, 0)),
        out_shape=jax.ShapeDtypeStruct((N_PAD, FW), jnp.float32),
    )(p0, p1, dvf, b_pad, a_prev.reshape(1, 1), W_pad)


def _tc_last(p0, p1, dvf, b4, batch_col, Wl, bl):
    C = Wl.shape[1]

    def body(p0_ref, p1_ref, dv_ref, b_ref, bat_ref, wl_ref, bl_ref,
             out_ref, sums, cnt):
        i = pl.program_id(0)

        @pl.when(i == 0)
        def _():
            sums[...] = jnp.zeros_like(sums)
            cnt[...] = jnp.zeros_like(cnt)

        h4 = dv_ref[...] * (p0_ref[...] + p1_ref[...]) + b_ref[...]
        seg = lax.broadcasted_iota(jnp.int32, (R, G), 1)
        onehot = (seg == bat_ref[...]).astype(jnp.float32)
        dn = (((0,), (0,)), ((), ()))
        sums[...] += lax.dot_general(onehot, h4, dn,
                                     preferred_element_type=jnp.float32)
        cnt[...] += lax.dot_general(onehot, jnp.full((R, 1), 1.0,
                                                     jnp.float32), dn,
                                    preferred_element_type=jnp.float32)

        @pl.when(i == NBLK - 1)
        def _():
            pooled = sums[...] / jnp.clip(cnt[...], 1.0, None)
            out_ref[...] = jnp.dot(pooled, wl_ref[...],
                                   preferred_element_type=jnp.float32) + bl_ref[...]

    return pl.pallas_call(
        body,
        grid=(NBLK,),
        in_specs=[
            pl.BlockSpec((R, FW), lambda i: (i, 0)),
            pl.BlockSpec((R, FW), lambda i: (i, 0)),
            pl.BlockSpec((R, FW), lambda i: (i, 0)),
            pl.BlockSpec((1, FW), lambda i: (0, 0)),
            pl.BlockSpec((R, 1), lambda i: (i, 0)),
            pl.BlockSpec(Wl.shape, lambda i: (0, 0)),
            pl.BlockSpec((1, C), lambda i: (0, 0)),
        ],
        out_specs=pl.BlockSpec((G, C), lambda i: (0, 0)),
        out_shape=jax.ShapeDtypeStruct((G, C), jnp.float32),
        scratch_shapes=[
            pltpu.VMEM((G, FW), jnp.float32),
            pltpu.VMEM((G, 1), jnp.float32),
        ],
    )(p0, p1, dvf, b4.reshape(1, FW), batch_col, Wl, bl.reshape(1, C))


def kernel(x, edge_index, batch, W1, b1, W2, b2, W3, b3, W4, b4,
           a1, a2, a3, Wl, bl):
    src = edge_index[0]
    dst = edge_index[1]
    x_pad = jnp.zeros((N_PAD, 128), jnp.float32).at[:N].set(x)
    batch_pad = jnp.full((N_PAD,), G, jnp.int32).at[:N].set(batch)
    batch_col = batch_pad.reshape(N_PAD, 1)

    d0, d1 = _deg_partials(dst)
    dvf, ht1 = _tc_first(x_pad, d0, d1, W1)

    p0, p1 = _agg_partials(ht1, src, dst)
    ht2 = _tc_mid(p0, p1, dvf, b1, a1, W2)

    p0, p1 = _agg_partials(ht2, src, dst)
    ht3 = _tc_mid(p0, p1, dvf, b2, a2, W3)

    p0, p1 = _agg_partials(ht3, src, dst)
    ht4 = _tc_mid(p0, p1, dvf, b3, a3, W4)

    p0, p1 = _agg_partials(ht4, src, dst)
    return _tc_last(p0, p1, dvf, b4, batch_col, Wl, bl)


# speedup vs baseline: 31.5831x; 1.5421x over previous
import functools

import jax
import jax.numpy as jnp
from jax import lax
from jax.experimental import pallas as pl
from jax.experimental.pallas import tpu as pltpu
from jax.experimental.pallas import tpu_sc as plsc

N = 10000
E = 320000
G = 64

NW = 32
EW = E // NW
N_PAD = 10240
ROWS_PT = N_PAD // 16
R = 512
NBLK = N_PAD // R

AGG_CFG = {
    16: (792, 12, (248, 248)),
    32: (536, 18, (176, 176)),
    64: (272, 36, (104, 104)),
    128: (120, 81, (120, 120, 40)),
}

_SC_PARAMS = pltpu.CompilerParams(use_tc_tiling_on_sc=False)


def _mesh():
    return plsc.VectorSubcoreMesh(core_axis_name="c", subcore_axis_name="s")


def _agg_partials(ht, src, dst, F):
    B, NFULL, TAILS = AGG_CFG[F]

    @functools.partial(
        pl.kernel,
        mesh=_mesh(),
        out_type=(jax.ShapeDtypeStruct((N_PAD, F), jnp.float32),
                  jax.ShapeDtypeStruct((N_PAD, F), jnp.float32)),
        scratch_types=[
            pltpu.VMEM((B,), jnp.int32),
            pltpu.VMEM((B,), jnp.int32),
            pltpu.VMEM((B, F), jnp.float32),
            pltpu.VMEM((B,), jnp.int32),
            pltpu.VMEM((B,), jnp.int32),
            pltpu.VMEM((B, F), jnp.float32),
            pltpu.VMEM((B,), jnp.int32),
            pltpu.VMEM((B,), jnp.int32),
            pltpu.VMEM((B, F), jnp.float32),
        ] + [pltpu.VMEM((sz,), jnp.int32) for sz in TAILS] * 2 + [
            pltpu.VMEM_SHARED((N_PAD, F), jnp.float32),
            pltpu.SemaphoreType.DMA,
            pltpu.SemaphoreType.DMA,
            pltpu.SemaphoreType.DMA,
            pltpu.SemaphoreType.DMA,
            pltpu.SemaphoreType.DMA,
            pltpu.SemaphoreType.DMA,
        ],
        compiler_params=_SC_PARAMS,
    )
    def k(h_hbm, src_hbm, dst_hbm, p0_hbm, p1_hbm, *refs):
        nt = len(TAILS)
        sidx = [refs[0], refs[3], refs[6]]
        didx = [refs[1], refs[4], refs[7]]
        rows = [refs[2], refs[5], refs[8]]
        sidx_t = list(refs[9:9 + nt])
        didx_t = list(refs[9 + nt:9 + 2 * nt])
        acc_sh = refs[9 + 2 * nt]
        gsem = list(refs[10 + 2 * nt:13 + 2 * nt])
        ssem = list(refs[13 + 2 * nt:16 + 2 * nt])

        c = lax.axis_index("c")
        s = lax.axis_index("s")
        w = s * 2 + c
        r0 = s * ROWS_PT

        @pl.when(c == 0)
        def _():
            pltpu.sync_copy(h_hbm.at[pl.ds(r0, ROWS_PT)],
                            acc_sh.at[pl.ds(r0, ROWS_PT)])

        @pl.when(c == 1)
        def _():
            nz = min(B, ROWS_PT)

            def zrow(i, _):
                for j in range(F // 16):
                    rows[0][i, pl.ds(j * 16, 16)] = jnp.zeros((16,),
                                                              jnp.float32)
                return 0

            lax.fori_loop(0, nz, zrow, 0)
            off = 0
            while off < ROWS_PT:
                sz = min(nz, ROWS_PT - off)
                pltpu.sync_copy(rows[0].at[pl.ds(0, sz)],
                                acc_sh.at[pl.ds(r0 + off, sz)])
                off += sz

        plsc.subcore_barrier()

        def phase_a(i, b, drain):
            if drain:
                pltpu.make_async_copy(rows[b], acc_sh.at[didx[b]],
                                      ssem[b]).wait()
            base = w * EW + i * B
            pltpu.sync_copy(src_hbm.at[pl.ds(base, B)], sidx[b])
            pltpu.sync_copy(dst_hbm.at[pl.ds(base, B)], didx[b])
            pltpu.async_copy(h_hbm.at[sidx[b]], rows[b], gsem[b])

        def phase_b(i, b):
            pltpu.make_async_copy(h_hbm.at[sidx[b]], rows[b],
                                  gsem[b]).wait()
            pltpu.async_copy(rows[b], acc_sh.at[didx[b]], ssem[b], add=True)

        phase_a(0, 0, False)
        phase_a(1, 1, False)
        phase_b(0, 0)
        phase_a(2, 2, False)
        phase_b(1, 1)

        def tri(m, _):
            i0 = 3 * m
            phase_a(i0, 0, True)
            phase_b(i0 - 1, 2)
            phase_a(i0 + 1, 1, True)
            phase_b(i0, 0)
            phase_a(i0 + 2, 2, True)
            phase_b(i0 + 1, 1)
            return 0

        lax.fori_loop(1, NFULL // 3, tri, 0)
        phase_b(NFULL - 1, (NFULL - 1) % 3)
        for b in range(3):
            pltpu.make_async_copy(rows[b], acc_sh.at[didx[b]],
                                  ssem[b]).wait()

        toff = NFULL * B
        for t, sz in enumerate(TAILS):
            base = w * EW + toff
            pltpu.sync_copy(src_hbm.at[pl.ds(base, sz)], sidx_t[t])
            pltpu.sync_copy(dst_hbm.at[pl.ds(base, sz)], didx_t[t])
            pltpu.async_copy(h_hbm.at[sidx_t[t]],
                             rows[t % 3].at[pl.ds(0, sz)], gsem[0]).wait()
            pltpu.sync_copy(rows[t % 3].at[pl.ds(0, sz)],
                            acc_sh.at[didx_t[t]], add=True)
            toff += sz
        plsc.subcore_barrier()

        @pl.when(c == 0)
        def _():
            pltpu.sync_copy(acc_sh.at[pl.ds(r0, ROWS_PT)],
                            p0_hbm.at[pl.ds(r0, ROWS_PT)])

        @pl.when(c == 1)
        def _():
            pltpu.sync_copy(acc_sh.at[pl.ds(r0, ROWS_PT)],
                            p1_hbm.at[pl.ds(r0, ROWS_PT)])

    return k(ht, src, dst)


FD = 16


def _deg_partials(dst):
    Bd = 1000
    nblk = EW // Bd

    @functools.partial(
        pl.kernel,
        mesh=_mesh(),
        out_type=(jax.ShapeDtypeStruct((N_PAD, FD), jnp.float32),
                  jax.ShapeDtypeStruct((N_PAD, FD), jnp.float32)),
        scratch_types=[
            pltpu.VMEM((Bd,), jnp.int32),
            pltpu.VMEM((Bd,), jnp.int32),
            pltpu.VMEM((Bd, FD), jnp.float32),
            pltpu.VMEM_SHARED((N_PAD, FD), jnp.float32),
            pltpu.SemaphoreType.DMA,
            pltpu.SemaphoreType.DMA,
        ],
        compiler_params=_SC_PARAMS,
    )
    def k(dst_hbm, p0_hbm, p1_hbm, idx0, idx1, ones_v, acc_sh,
          ssem0, ssem1):
        c = lax.axis_index("c")
        s = lax.axis_index("s")
        w = s * 2 + c
        r0 = s * ROWS_PT
        idx = [idx0, idx1]
        ssem = [ssem0, ssem1]

        def fill(i, _):
            for j in range(FD // 16):
                ones_v[i, pl.ds(j * 16, 16)] = jnp.zeros((16,), jnp.float32)
            return 0

        lax.fori_loop(0, Bd, fill, 0)
        off = 0
        while off < ROWS_PT:
            sz = min(Bd, ROWS_PT - off)
            pltpu.sync_copy(ones_v.at[pl.ds(0, sz)],
                            acc_sh.at[pl.ds(r0 + off, sz)])
            off += sz

        def fill1(i, _):
            for j in range(FD // 16):
                ones_v[i, pl.ds(j * 16, 16)] = jnp.full((16,), 1.0,
                                                        jnp.float32)
            return 0

        lax.fori_loop(0, Bd, fill1, 0)
        plsc.subcore_barrier()

        def do_block(i, b, first):
            if not first:
                pltpu.make_async_copy(ones_v, acc_sh.at[idx[b]],
                                      ssem[b]).wait()
            pltpu.sync_copy(dst_hbm.at[pl.ds(w * EW + i * Bd, Bd)], idx[b])
            pltpu.async_copy(ones_v, acc_sh.at[idx[b]], ssem[b], add=True)

        do_block(0, 0, True)
        do_block(1, 1, True)

        def pair(j, _):
            do_block(2 * j, 0, False)
            do_block(2 * j + 1, 1, False)
            return 0

        lax.fori_loop(1, nblk // 2, pair, 0)
        pltpu.make_async_copy(ones_v, acc_sh.at[idx0], ssem0).wait()
        pltpu.make_async_copy(ones_v, acc_sh.at[idx1], ssem1).wait()
        plsc.subcore_barrier()

        @pl.when(c == 0)
        def _():
            pltpu.sync_copy(acc_sh.at[pl.ds(r0, ROWS_PT)],
                            p0_hbm.at[pl.ds(r0, ROWS_PT)])

        @pl.when(c == 1)
        def _():
            pltpu.sync_copy(acc_sh.at[pl.ds(r0, ROWS_PT)],
                            p1_hbm.at[pl.ds(r0, ROWS_PT)])

    return k(dst)


def _bcast_col(dv, width):
    return jnp.broadcast_to(dv[:, :1], (dv.shape[0], width))


def _tc_first(x_pad, d0, d1, W1):
    F = W1.shape[1]

    def body(x_ref, d0_ref, d1_ref, w_ref, dv_ref, ht_ref):
        deg = d0_ref[...] + d1_ref[...] + 1.0
        dv = lax.rsqrt(deg)
        dv_ref[...] = dv
        ht = jnp.dot(x_ref[...], w_ref[...],
                     preferred_element_type=jnp.float32)
        ht_ref[...] = ht * _bcast_col(dv, F)

    return pl.pallas_call(
        body,
        grid=(NBLK,),
        in_specs=[
            pl.BlockSpec((R, 128), lambda i: (i, 0)),
            pl.BlockSpec((R, FD), lambda i: (i, 0)),
            pl.BlockSpec((R, FD), lambda i: (i, 0)),
            pl.BlockSpec(W1.shape, lambda i: (0, 0)),
        ],
        out_specs=[
            pl.BlockSpec((R, FD), lambda i: (i, 0)),
            pl.BlockSpec((R, F), lambda i: (i, 0)),
        ],
        out_shape=[
            jax.ShapeDtypeStruct((N_PAD, FD), jnp.float32),
            jax.ShapeDtypeStruct((N_PAD, F), jnp.float32),
        ],
    )(x_pad, d0, d1, W1)


def _tc_mid(p0, p1, dvf, b_prev, a_prev, W_next):
    Fp = W_next.shape[0]
    Fn = W_next.shape[1]

    def body(p0_ref, p1_ref, dv_ref, b_ref, a_ref, w_ref, ht_ref):
        dv = dv_ref[...]
        t = _bcast_col(dv, Fp) * (p0_ref[...] + p1_ref[...]) + b_ref[...]
        a = a_ref[0, 0]
        h = jnp.where(t >= 0, t, a * t)
        ht = jnp.dot(h, w_ref[...], preferred_element_type=jnp.float32)
        ht_ref[...] = ht * _bcast_col(dv, Fn)

    return pl.pallas_call(
        body,
        grid=(NBLK,),
        in_specs=[
            pl.BlockSpec((R, Fp), lambda i: (i, 0)),
            pl.BlockSpec((R, Fp), lambda i: (i, 0)),
            pl.BlockSpec((R, FD), lambda i: (i, 0)),
            pl.BlockSpec((1, Fp), lambda i: (0, 0)),
            pl.BlockSpec(memory_space=pltpu.SMEM),
            pl.BlockSpec((Fp, Fn), lambda i: (0, 0)),
        ],
        out_specs=pl.BlockSpec((R, Fn), lambda i: (i, 0)),
        out_shape=jax.ShapeDtypeStruct((N_PAD, Fn), jnp.float32),
    )(p0, p1, dvf, b_prev.reshape(1, Fp), a_prev.reshape(1, 1), W_next)


def _tc_last(p0, p1, dvf, b4, batch_col, Wl, bl):
    C = Wl.shape[1]
    F4 = Wl.shape[0]

    def body(p0_ref, p1_ref, dv_ref, b_ref, bat_ref, wl_ref, bl_ref,
             out_ref, sums, cnt):
        i = pl.program_id(0)

        @pl.when(i == 0)
        def _():
            sums[...] = jnp.zeros_like(sums)
            cnt[...] = jnp.zeros_like(cnt)

        h4 = (_bcast_col(dv_ref[...], F4) * (p0_ref[...] + p1_ref[...])
              + b_ref[...])
        seg = lax.broadcasted_iota(jnp.int32, (R, G), 1)
        onehot = (seg == bat_ref[...]).astype(jnp.float32)
        dn = (((0,), (0,)), ((), ()))
        sums[...] += lax.dot_general(onehot, h4, dn,
                                     preferred_element_type=jnp.float32)
        cnt[...] += lax.dot_general(onehot, jnp.full((R, 1), 1.0,
                                                     jnp.float32), dn,
                                    preferred_element_type=jnp.float32)

        @pl.when(i == NBLK - 1)
        def _():
            pooled = sums[...] / jnp.clip(cnt[...], 1.0, None)
            out_ref[...] = jnp.dot(pooled, wl_ref[...],
                                   preferred_element_type=jnp.float32) + bl_ref[...]

    return pl.pallas_call(
        body,
        grid=(NBLK,),
        in_specs=[
            pl.BlockSpec((R, F4), lambda i: (i, 0)),
            pl.BlockSpec((R, F4), lambda i: (i, 0)),
            pl.BlockSpec((R, FD), lambda i: (i, 0)),
            pl.BlockSpec((1, F4), lambda i: (0, 0)),
            pl.BlockSpec((R, 1), lambda i: (i, 0)),
            pl.BlockSpec(Wl.shape, lambda i: (0, 0)),
            pl.BlockSpec((1, C), lambda i: (0, 0)),
        ],
        out_specs=pl.BlockSpec((G, C), lambda i: (0, 0)),
        out_shape=jax.ShapeDtypeStruct((G, C), jnp.float32),
        scratch_shapes=[
            pltpu.VMEM((G, F4), jnp.float32),
            pltpu.VMEM((G, 1), jnp.float32),
        ],
    )(p0, p1, dvf, b4.reshape(1, F4), batch_col, Wl, bl.reshape(1, C))


def kernel(x, edge_index, batch, W1, b1, W2, b2, W3, b3, W4, b4,
           a1, a2, a3, Wl, bl):
    src = edge_index[0]
    dst = edge_index[1]
    x_pad = jnp.zeros((N_PAD, 128), jnp.float32).at[:N].set(x)
    batch_pad = jnp.full((N_PAD,), G, jnp.int32).at[:N].set(batch)
    batch_col = batch_pad.reshape(N_PAD, 1)

    d0, d1 = _deg_partials(dst)
    dvf, ht1 = _tc_first(x_pad, d0, d1, W1)

    p0, p1 = _agg_partials(ht1, src, dst, 16)
    ht2 = _tc_mid(p0, p1, dvf, b1, a1, W2)

    p0, p1 = _agg_partials(ht2, src, dst, 32)
    ht3 = _tc_mid(p0, p1, dvf, b2, a2, W3)

    p0, p1 = _agg_partials(ht3, src, dst, 64)
    ht4 = _tc_mid(p0, p1, dvf, b3, a3, W4)

    p0, p1 = _agg_partials(ht4, src, dst, 128)
    return _tc_last(p0, p1, dvf, b4, batch_col, Wl, bl)


# speedup vs baseline: 34.5022x; 1.0924x over previous
import functools

import jax
import jax.numpy as jnp
from jax import lax
from jax.experimental import pallas as pl
from jax.experimental.pallas import tpu as pltpu
from jax.experimental.pallas import tpu_sc as plsc

N = 10000
E = 320000
G = 64

NW = 32
EW = E // NW
N_PAD = 10240
ROWS_PT = N_PAD // 16
R = 2048
NBLK = N_PAD // R

AGG_CFG = {
    16: (792, 12, (248, 248)),
    32: (536, 18, (176, 176)),
    64: (272, 36, (104, 104)),
    128: (120, 81, (120, 120, 40)),
}

_SC_PARAMS = pltpu.CompilerParams(use_tc_tiling_on_sc=False)


def _mesh():
    return plsc.VectorSubcoreMesh(core_axis_name="c", subcore_axis_name="s")


def _agg_partials(ht, src, dst, F):
    B, NFULL, TAILS = AGG_CFG[F]

    @functools.partial(
        pl.kernel,
        mesh=_mesh(),
        out_type=jax.ShapeDtypeStruct((2, N_PAD, F), jnp.float32),
        scratch_types=[
            pltpu.VMEM((B,), jnp.int32),
            pltpu.VMEM((B,), jnp.int32),
            pltpu.VMEM((B, F), jnp.float32),
            pltpu.VMEM((B,), jnp.int32),
            pltpu.VMEM((B,), jnp.int32),
            pltpu.VMEM((B, F), jnp.float32),
            pltpu.VMEM((B,), jnp.int32),
            pltpu.VMEM((B,), jnp.int32),
            pltpu.VMEM((B, F), jnp.float32),
        ] + [pltpu.VMEM((sz,), jnp.int32) for sz in TAILS] * 2 + [
            pltpu.VMEM_SHARED((N_PAD, F), jnp.float32),
            pltpu.SemaphoreType.DMA,
            pltpu.SemaphoreType.DMA,
            pltpu.SemaphoreType.DMA,
            pltpu.SemaphoreType.DMA,
            pltpu.SemaphoreType.DMA,
            pltpu.SemaphoreType.DMA,
        ],
        compiler_params=_SC_PARAMS,
    )
    def k(h_hbm, src_hbm, dst_hbm, p_hbm, *refs):
        nt = len(TAILS)
        sidx = [refs[0], refs[3], refs[6]]
        didx = [refs[1], refs[4], refs[7]]
        rows = [refs[2], refs[5], refs[8]]
        sidx_t = list(refs[9:9 + nt])
        didx_t = list(refs[9 + nt:9 + 2 * nt])
        acc_sh = refs[9 + 2 * nt]
        gsem = list(refs[10 + 2 * nt:13 + 2 * nt])
        ssem = list(refs[13 + 2 * nt:16 + 2 * nt])

        c = lax.axis_index("c")
        s = lax.axis_index("s")
        w = s * 2 + c
        r0 = s * ROWS_PT

        @pl.when(c == 0)
        def _():
            pltpu.sync_copy(h_hbm.at[pl.ds(r0, ROWS_PT)],
                            acc_sh.at[pl.ds(r0, ROWS_PT)])

        @pl.when(c == 1)
        def _():
            nz = min(B, ROWS_PT)

            def zrow(i, _):
                for j in range(F // 16):
                    rows[0][i, pl.ds(j * 16, 16)] = jnp.zeros((16,),
                                                              jnp.float32)
                return 0

            lax.fori_loop(0, nz, zrow, 0)
            off = 0
            while off < ROWS_PT:
                sz = min(nz, ROWS_PT - off)
                pltpu.sync_copy(rows[0].at[pl.ds(0, sz)],
                                acc_sh.at[pl.ds(r0 + off, sz)])
                off += sz

        plsc.subcore_barrier()

        def phase_a(i, b, drain):
            if drain:
                pltpu.make_async_copy(rows[b], acc_sh.at[didx[b]],
                                      ssem[b]).wait()
            base = w * EW + i * B
            pltpu.sync_copy(src_hbm.at[pl.ds(base, B)], sidx[b])
            pltpu.sync_copy(dst_hbm.at[pl.ds(base, B)], didx[b])
            pltpu.async_copy(h_hbm.at[sidx[b]], rows[b], gsem[b])

        def phase_b(i, b):
            pltpu.make_async_copy(h_hbm.at[sidx[b]], rows[b],
                                  gsem[b]).wait()
            pltpu.async_copy(rows[b], acc_sh.at[didx[b]], ssem[b], add=True)

        phase_a(0, 0, False)
        phase_a(1, 1, False)
        phase_b(0, 0)
        phase_a(2, 2, False)
        phase_b(1, 1)

        def tri(m, _):
            i0 = 3 * m
            phase_a(i0, 0, True)
            phase_b(i0 - 1, 2)
            phase_a(i0 + 1, 1, True)
            phase_b(i0, 0)
            phase_a(i0 + 2, 2, True)
            phase_b(i0 + 1, 1)
            return 0

        lax.fori_loop(1, NFULL // 3, tri, 0)
        phase_b(NFULL - 1, (NFULL - 1) % 3)
        for b in range(3):
            pltpu.make_async_copy(rows[b], acc_sh.at[didx[b]],
                                  ssem[b]).wait()

        toff = NFULL * B
        for t, sz in enumerate(TAILS):
            base = w * EW + toff
            pltpu.sync_copy(src_hbm.at[pl.ds(base, sz)], sidx_t[t])
            pltpu.sync_copy(dst_hbm.at[pl.ds(base, sz)], didx_t[t])
            pltpu.async_copy(h_hbm.at[sidx_t[t]],
                             rows[t % 3].at[pl.ds(0, sz)], gsem[0]).wait()
            pltpu.sync_copy(rows[t % 3].at[pl.ds(0, sz)],
                            acc_sh.at[didx_t[t]], add=True)
            toff += sz
        plsc.subcore_barrier()

        @pl.when(c == 0)
        def _():
            pltpu.sync_copy(acc_sh.at[pl.ds(r0, ROWS_PT)],
                            p_hbm.at[0, pl.ds(r0, ROWS_PT)])

        @pl.when(c == 1)
        def _():
            pltpu.sync_copy(acc_sh.at[pl.ds(r0, ROWS_PT)],
                            p_hbm.at[1, pl.ds(r0, ROWS_PT)])

    return k(ht, src, dst)


FD = 16


def _deg_partials(dst):
    Bd = 1000
    nblk = EW // Bd

    @functools.partial(
        pl.kernel,
        mesh=_mesh(),
        out_type=jax.ShapeDtypeStruct((2, N_PAD, FD), jnp.float32),
        scratch_types=[
            pltpu.VMEM((Bd,), jnp.int32),
            pltpu.VMEM((Bd,), jnp.int32),
            pltpu.VMEM((Bd, FD), jnp.float32),
            pltpu.VMEM_SHARED((N_PAD, FD), jnp.float32),
            pltpu.SemaphoreType.DMA,
            pltpu.SemaphoreType.DMA,
        ],
        compiler_params=_SC_PARAMS,
    )
    def k(dst_hbm, p_hbm, idx0, idx1, ones_v, acc_sh,
          ssem0, ssem1):
        c = lax.axis_index("c")
        s = lax.axis_index("s")
        w = s * 2 + c
        r0 = s * ROWS_PT
        idx = [idx0, idx1]
        ssem = [ssem0, ssem1]

        def fill(i, _):
            for j in range(FD // 16):
                ones_v[i, pl.ds(j * 16, 16)] = jnp.zeros((16,), jnp.float32)
            return 0

        lax.fori_loop(0, Bd, fill, 0)
        off = 0
        while off < ROWS_PT:
            sz = min(Bd, ROWS_PT - off)
            pltpu.sync_copy(ones_v.at[pl.ds(0, sz)],
                            acc_sh.at[pl.ds(r0 + off, sz)])
            off += sz

        def fill1(i, _):
            for j in range(FD // 16):
                ones_v[i, pl.ds(j * 16, 16)] = jnp.full((16,), 1.0,
                                                        jnp.float32)
            return 0

        lax.fori_loop(0, Bd, fill1, 0)
        plsc.subcore_barrier()

        def do_block(i, b, first):
            if not first:
                pltpu.make_async_copy(ones_v, acc_sh.at[idx[b]],
                                      ssem[b]).wait()
            pltpu.sync_copy(dst_hbm.at[pl.ds(w * EW + i * Bd, Bd)], idx[b])
            pltpu.async_copy(ones_v, acc_sh.at[idx[b]], ssem[b], add=True)

        do_block(0, 0, True)
        do_block(1, 1, True)

        def pair(j, _):
            do_block(2 * j, 0, False)
            do_block(2 * j + 1, 1, False)
            return 0

        lax.fori_loop(1, nblk // 2, pair, 0)
        pltpu.make_async_copy(ones_v, acc_sh.at[idx0], ssem0).wait()
        pltpu.make_async_copy(ones_v, acc_sh.at[idx1], ssem1).wait()
        plsc.subcore_barrier()

        @pl.when(c == 0)
        def _():
            pltpu.sync_copy(acc_sh.at[pl.ds(r0, ROWS_PT)],
                            p_hbm.at[0, pl.ds(r0, ROWS_PT)])

        @pl.when(c == 1)
        def _():
            pltpu.sync_copy(acc_sh.at[pl.ds(r0, ROWS_PT)],
                            p_hbm.at[1, pl.ds(r0, ROWS_PT)])

    return k(dst)


def _bcast_col(dv, width):
    return jnp.broadcast_to(dv[:, :1], (dv.shape[0], width))


def _tc_first(x_pad, d, W1):
    F = W1.shape[1]

    def body(x_ref, d_ref, w_ref, dv_ref, ht_ref):
        deg = d_ref[0] + d_ref[1] + 1.0
        dv = lax.rsqrt(deg)
        dv_ref[...] = dv
        ht = jnp.dot(x_ref[...], w_ref[...],
                     preferred_element_type=jnp.float32)
        ht_ref[...] = ht * _bcast_col(dv, F)

    return pl.pallas_call(
        body,
        grid=(NBLK,),
        in_specs=[
            pl.BlockSpec((R, 128), lambda i: (i, 0)),
            pl.BlockSpec((2, R, FD), lambda i: (0, i, 0)),
            pl.BlockSpec(W1.shape, lambda i: (0, 0)),
        ],
        out_specs=[
            pl.BlockSpec((R, FD), lambda i: (i, 0)),
            pl.BlockSpec((R, F), lambda i: (i, 0)),
        ],
        out_shape=[
            jax.ShapeDtypeStruct((N_PAD, FD), jnp.float32),
            jax.ShapeDtypeStruct((N_PAD, F), jnp.float32),
        ],
    )(x_pad, d, W1)


def _tc_mid(p, dvf, b_prev, a_prev, W_next):
    Fp = W_next.shape[0]
    Fn = W_next.shape[1]

    def body(p_ref, dv_ref, b_ref, a_ref, w_ref, ht_ref):
        dv = dv_ref[...]
        t = _bcast_col(dv, Fp) * (p_ref[0] + p_ref[1]) + b_ref[...]
        a = a_ref[0, 0]
        h = jnp.where(t >= 0, t, a * t)
        ht = jnp.dot(h, w_ref[...], preferred_element_type=jnp.float32)
        ht_ref[...] = ht * _bcast_col(dv, Fn)

    return pl.pallas_call(
        body,
        grid=(NBLK,),
        in_specs=[
            pl.BlockSpec((2, R, Fp), lambda i: (0, i, 0)),
            pl.BlockSpec((R, FD), lambda i: (i, 0)),
            pl.BlockSpec((1, Fp), lambda i: (0, 0)),
            pl.BlockSpec(memory_space=pltpu.SMEM),
            pl.BlockSpec((Fp, Fn), lambda i: (0, 0)),
        ],
        out_specs=pl.BlockSpec((R, Fn), lambda i: (i, 0)),
        out_shape=jax.ShapeDtypeStruct((N_PAD, Fn), jnp.float32),
    )(p, dvf, b_prev.reshape(1, Fp), a_prev.reshape(1, 1), W_next)


def _tc_last(p, dvf, b4, batch_col, Wl, bl):
    C = Wl.shape[1]
    F4 = Wl.shape[0]

    def body(p_ref, dv_ref, b_ref, bat_ref, wl_ref, bl_ref,
             out_ref, sums, cnt):
        i = pl.program_id(0)

        @pl.when(i == 0)
        def _():
            sums[...] = jnp.zeros_like(sums)
            cnt[...] = jnp.zeros_like(cnt)

        h4 = (_bcast_col(dv_ref[...], F4) * (p_ref[0] + p_ref[1])
              + b_ref[...])
        seg = lax.broadcasted_iota(jnp.int32, (R, G), 1)
        onehot = (seg == bat_ref[...]).astype(jnp.float32)
        dn = (((0,), (0,)), ((), ()))
        sums[...] += lax.dot_general(onehot, h4, dn,
                                     preferred_element_type=jnp.float32)
        cnt[...] += lax.dot_general(onehot, jnp.full((R, 1), 1.0,
                                                     jnp.float32), dn,
                                    preferred_element_type=jnp.float32)

        @pl.when(i == NBLK - 1)
        def _():
            pooled = sums[...] / jnp.clip(cnt[...], 1.0, None)
            out_ref[...] = jnp.dot(pooled, wl_ref[...],
                                   preferred_element_type=jnp.float32) + bl_ref[...]

    return pl.pallas_call(
        body,
        grid=(NBLK,),
        in_specs=[
            pl.BlockSpec((2, R, F4), lambda i: (0, i, 0)),
            pl.BlockSpec((R, FD), lambda i: (i, 0)),
            pl.BlockSpec((1, F4), lambda i: (0, 0)),
            pl.BlockSpec((R, 1), lambda i: (i, 0)),
            pl.BlockSpec(Wl.shape, lambda i: (0, 0)),
            pl.BlockSpec((1, C), lambda i: (0, 0)),
        ],
        out_specs=pl.BlockSpec((G, C), lambda i: (0, 0)),
        out_shape=jax.ShapeDtypeStruct((G, C), jnp.float32),
        scratch_shapes=[
            pltpu.VMEM((G, F4), jnp.float32),
            pltpu.VMEM((G, 1), jnp.float32),
        ],
    )(p, dvf, b4.reshape(1, F4), batch_col, Wl, bl.reshape(1, C))


def kernel(x, edge_index, batch, W1, b1, W2, b2, W3, b3, W4, b4,
           a1, a2, a3, Wl, bl):
    src = edge_index[0]
    dst = edge_index[1]
    x_pad = jnp.pad(x, ((0, N_PAD - N), (0, 0)))
    batch_pad = jnp.pad(batch, (0, N_PAD - N), constant_values=G)
    batch_col = batch_pad.reshape(N_PAD, 1)

    d = _deg_partials(dst)
    dvf, ht1 = _tc_first(x_pad, d, W1)

    p = _agg_partials(ht1, src, dst, 16)
    ht2 = _tc_mid(p, dvf, b1, a1, W2)

    p = _agg_partials(ht2, src, dst, 32)
    ht3 = _tc_mid(p, dvf, b2, a2, W3)

    p = _agg_partials(ht3, src, dst, 64)
    ht4 = _tc_mid(p, dvf, b3, a3, W4)

    p = _agg_partials(ht4, src, dst, 128)
    return _tc_last(p, dvf, b4, batch_col, Wl, bl)
